# Initial kernel scaffold; baseline (speedup 1.0000x reference)
import jax
import jax.numpy as jnp
from jax.experimental import pallas as pl


def kernel(x, edge_index, epsilon, W1, b1, g1, be1, W2, b2, g2, be2):
    raise NotImplementedError("write your pallas kernel here")


# speedup vs baseline: 5.5155x; 5.5155x over previous
import functools

import jax
import jax.numpy as jnp
from jax import lax
from jax.experimental import pallas as pl
from jax.experimental.pallas import tpu as pltpu
from jax.experimental.pallas import tpu_sc as plsc

N_NODES = 10000
N_EDGES = 320000
D = 128

NC = 2
NS = 16
NW = NC * NS

EDGES_PER_TILE = N_EDGES // NW
CHUNK = 80
N_CHUNKS = EDGES_PER_TILE // CHUNK
ACC_ROWS = 10240
TILE_STRIPE = ACC_ROWS // NS
STAGE = 128


def _sc_aggregate(x, src, dst):
    mesh = plsc.VectorSubcoreMesh(core_axis_name="c", subcore_axis_name="s")

    @functools.partial(
        pl.kernel,
        mesh=mesh,
        out_type=jax.ShapeDtypeStruct((NC * ACC_ROWS, D), jnp.float32),
        scratch_types=[
            pltpu.VMEM((CHUNK,), jnp.int32),
            pltpu.VMEM((CHUNK,), jnp.int32),
            pltpu.VMEM((CHUNK, D), jnp.float32),
            pltpu.VMEM((STAGE, D), jnp.float32),
            pltpu.VMEM_SHARED((ACC_ROWS, D), jnp.float32),
            pltpu.SemaphoreType.DMA,
        ],
    )
    def agg(x_hbm, src_hbm, dst_hbm, out_hbm, sidx, didx, rows, stage, acc, sem):
        cid = lax.axis_index("c")
        sid = lax.axis_index("s")
        wid = cid * NS + sid

        def zrow(r, carry):
            for c16 in range(D // 16):
                stage[r, pl.ds(c16 * 16, 16)] = jnp.zeros((16,), jnp.float32)
            return carry

        lax.fori_loop(0, STAGE, zrow, 0)
        base_row = sid * TILE_STRIPE
        for j in range(TILE_STRIPE // STAGE):
            pltpu.sync_copy(stage, acc.at[pl.ds(base_row + j * STAGE, STAGE)])
        plsc.subcore_barrier()

        ebase = wid * EDGES_PER_TILE

        def chunk_body(c, carry):
            off = pl.multiple_of(ebase + c * CHUNK, 8)
            pltpu.sync_copy(src_hbm.at[pl.ds(off, CHUNK)], sidx)
            pltpu.sync_copy(dst_hbm.at[pl.ds(off, CHUNK)], didx)
            pltpu.async_copy(x_hbm.at[sidx], rows, sem).wait()
            pltpu.sync_copy(rows, acc.at[didx], add=True)
            return carry

        lax.fori_loop(0, N_CHUNKS, chunk_body, 0)
        plsc.subcore_barrier()

        out_base = cid * ACC_ROWS + base_row
        for j in range(TILE_STRIPE // STAGE):
            pltpu.sync_copy(acc.at[pl.ds(base_row + j * STAGE, STAGE)], stage)
            pltpu.sync_copy(stage, out_hbm.at[pl.ds(out_base + j * STAGE, STAGE)])

    return agg(x, src, dst)


def _tc_mlp(partials, x, epsilon, W1, b1, g1, be1, W2, b2, g2, be2):
    def body(p_ref, x_ref, eps_ref, W1_ref, b1_ref, g1_ref, be1_ref,
             W2_ref, b2_ref, g2_ref, be2_ref, out_ref):
        v = (p_ref[0, 0:N_NODES, :] + p_ref[1, 0:N_NODES, :]
             + eps_ref[0, 0] * x_ref[...])
        h = jnp.dot(v, W1_ref[...], preferred_element_type=jnp.float32) + b1_ref[...]
        m = jnp.mean(h, axis=0)
        var = jnp.mean((h - m) * (h - m), axis=0)
        h = jnp.maximum((h - m) * lax.rsqrt(var + 1e-5) * g1_ref[...] + be1_ref[...], 0.0)
        h = jnp.dot(h, W2_ref[...], preferred_element_type=jnp.float32) + b2_ref[...]
        m2 = jnp.mean(h, axis=0)
        var2 = jnp.mean((h - m2) * (h - m2), axis=0)
        out_ref[...] = jnp.maximum(
            (h - m2) * lax.rsqrt(var2 + 1e-5) * g2_ref[...] + be2_ref[...], 0.0)

    return pl.pallas_call(
        body,
        out_shape=jax.ShapeDtypeStruct((N_NODES, D), jnp.float32),
    )(partials, x, epsilon, W1, b1, g1, be1, W2, b2, g2, be2)


def kernel(x, edge_index, epsilon, W1, b1, g1, be1, W2, b2, g2, be2):
    src = edge_index[0]
    dst = edge_index[1]
    partials = _sc_aggregate(x, src, dst)
    partials = partials.reshape(NC, ACC_ROWS, D)
    return _tc_mlp(partials, x, epsilon, W1, b1, g1, be1, W2, b2, g2, be2)


# speedup vs baseline: 10.6812x; 1.9366x over previous
import functools

import jax
import jax.numpy as jnp
from jax import lax
from jax.experimental import pallas as pl
from jax.experimental.pallas import tpu as pltpu
from jax.experimental.pallas import tpu_sc as plsc

N_NODES = 10000
N_EDGES = 320000
D = 128

NC = 2
NS = 16
NW = NC * NS

CHUNK = 128
N_CHUNKS = N_EDGES // CHUNK
BASE_CHUNKS = N_CHUNKS // NW
EXTRA_TILES = N_CHUNKS - BASE_CHUNKS * NW
NBUF = 3
N_GROUPS = BASE_CHUNKS // NBUF
ACC_ROWS = 10112
TILE_STRIPE = ACC_ROWS // NS
STRIPE_COPIES = (128, 128, 128, 128, 120)


def _sc_aggregate(x, src, dst):
    mesh = plsc.VectorSubcoreMesh(core_axis_name="c", subcore_axis_name="s")

    @functools.partial(
        pl.kernel,
        mesh=mesh,
        out_type=jax.ShapeDtypeStruct((NC * ACC_ROWS, D), jnp.float32),
        scratch_types=[pltpu.VMEM((CHUNK,), jnp.int32) for _ in range(NBUF)]
        + [pltpu.VMEM((CHUNK,), jnp.int32) for _ in range(NBUF)]
        + [pltpu.VMEM((CHUNK, D), jnp.float32) for _ in range(NBUF)]
        + [pltpu.VMEM_SHARED((ACC_ROWS, D), jnp.float32)]
        + [pltpu.SemaphoreType.DMA for _ in range(3 * NBUF)],
    )
    def agg(x_hbm, src_hbm, dst_hbm, out_hbm, *rest):
        sidx = rest[:NBUF]
        didx = rest[NBUF:2 * NBUF]
        rows = rest[2 * NBUF:3 * NBUF]
        acc = rest[3 * NBUF]
        ssem = rest[3 * NBUF + 1:4 * NBUF + 1]
        dsem = rest[4 * NBUF + 1:5 * NBUF + 1]
        gsem = rest[5 * NBUF + 1:6 * NBUF + 1]
        cid = lax.axis_index("c")
        sid = lax.axis_index("s")
        wid = cid * NS + sid

        def zrow(r, carry):
            for c16 in range(D // 16):
                rows[0][r, pl.ds(c16 * 16, 16)] = jnp.zeros((16,), jnp.float32)
            return carry

        lax.fori_loop(0, CHUNK, zrow, 0)
        base_row = sid * TILE_STRIPE
        off = 0
        for n in STRIPE_COPIES:
            pltpu.sync_copy(rows[0].at[pl.ds(0, n)],
                            acc.at[pl.ds(base_row + off, n)])
            off += n
        plsc.subcore_barrier()

        chunk0 = wid * BASE_CHUNKS + jnp.minimum(wid, EXTRA_TILES)

        def issue_idx(b, c):
            eoff = pl.multiple_of((chunk0 + c) * CHUNK, 8)
            pltpu.async_copy(src_hbm.at[pl.ds(eoff, CHUNK)], sidx[b], ssem[b])
            pltpu.async_copy(dst_hbm.at[pl.ds(eoff, CHUNK)], didx[b], dsem[b])

        def wait_src(b):
            pltpu.make_async_copy(src_hbm.at[pl.ds(0, CHUNK)], sidx[b], ssem[b]).wait()

        def wait_dst(b):
            pltpu.make_async_copy(dst_hbm.at[pl.ds(0, CHUNK)], didx[b], dsem[b]).wait()

        def issue_gather(b):
            pltpu.async_copy(x_hbm.at[sidx[b]], rows[b], gsem[b])

        def wait_gather(b):
            pltpu.make_async_copy(x_hbm.at[sidx[b]], rows[b], gsem[b]).wait()

        for b in range(NBUF):
            issue_idx(b, b)
        for b in range(NBUF):
            wait_src(b)
            issue_gather(b)

        def group(g, carry):
            for b in range(NBUF):
                wait_gather(b)
                wait_dst(b)
                pltpu.sync_copy(rows[b], acc.at[didx[b]], add=True)

                @pl.when(g < N_GROUPS - 1)
                def _():
                    issue_idx(b, g * NBUF + b + NBUF)

            for b in range(NBUF):
                @pl.when(g < N_GROUPS - 1)
                def _():
                    wait_src(b)
                    issue_gather(b)
            return carry

        lax.fori_loop(0, N_GROUPS, group, 0)

        @pl.when(wid < EXTRA_TILES)
        def _():
            issue_idx(0, BASE_CHUNKS)
            wait_src(0)
            issue_gather(0)
            wait_gather(0)
            wait_dst(0)
            pltpu.sync_copy(rows[0], acc.at[didx[0]], add=True)

        plsc.subcore_barrier()

        out_base = cid * ACC_ROWS + base_row
        off = 0
        for n in STRIPE_COPIES:
            pltpu.sync_copy(acc.at[pl.ds(base_row + off, n)], rows[0].at[pl.ds(0, n)])
            pltpu.sync_copy(rows[0].at[pl.ds(0, n)], out_hbm.at[pl.ds(out_base + off, n)])
            off += n

    return agg(x, src, dst)


def _tc_mlp(partials, x, epsilon, W1, b1, g1, be1, W2, b2, g2, be2):
    def body(p_ref, x_ref, eps_ref, W1_ref, b1_ref, g1_ref, be1_ref,
             W2_ref, b2_ref, g2_ref, be2_ref, out_ref):
        v = (p_ref[0:N_NODES, :] + p_ref[ACC_ROWS:ACC_ROWS + N_NODES, :]
             + eps_ref[0, 0] * x_ref[...])
        h = jnp.dot(v, W1_ref[...], preferred_element_type=jnp.float32) + b1_ref[...]
        m = jnp.mean(h, axis=0)
        var = jnp.mean((h - m) * (h - m), axis=0)
        h = jnp.maximum((h - m) * lax.rsqrt(var + 1e-5) * g1_ref[...] + be1_ref[...], 0.0)
        h = jnp.dot(h, W2_ref[...], preferred_element_type=jnp.float32) + b2_ref[...]
        m2 = jnp.mean(h, axis=0)
        var2 = jnp.mean((h - m2) * (h - m2), axis=0)
        out_ref[...] = jnp.maximum(
            (h - m2) * lax.rsqrt(var2 + 1e-5) * g2_ref[...] + be2_ref[...], 0.0)

    return pl.pallas_call(
        body,
        out_shape=jax.ShapeDtypeStruct((N_NODES, D), jnp.float32),
    )(partials, x, epsilon, W1, b1, g1, be1, W2, b2, g2, be2)


def kernel(x, edge_index, epsilon, W1, b1, g1, be1, W2, b2, g2, be2):
    src = edge_index[0]
    dst = edge_index[1]
    partials = _sc_aggregate(x, src, dst)
    return _tc_mlp(partials, x, epsilon, W1, b1, g1, be1, W2, b2, g2, be2)


# speedup vs baseline: 14.1027x; 1.3203x over previous
import functools

import jax
import jax.numpy as jnp
from jax import lax
from jax.experimental import pallas as pl
from jax.experimental.pallas import tpu as pltpu
from jax.experimental.pallas import tpu_sc as plsc

N_NODES = 10000
N_EDGES = 320000
D = 128

NC = 2
NS = 16
NW = NC * NS

CHUNK = 128
N_CHUNKS = N_EDGES // CHUNK
BASE_CHUNKS = N_CHUNKS // NW
EXTRA_TILES = N_CHUNKS - BASE_CHUNKS * NW
NBUF = 3
N_GROUPS = BASE_CHUNKS // NBUF
ACC_ROWS = 10112
TILE_STRIPE = ACC_ROWS // NS
STRIPE_COPIES = (128, 128, 128, 128, 120)


def _sc_aggregate(x, src, dst):
    mesh = plsc.VectorSubcoreMesh(core_axis_name="c", subcore_axis_name="s")

    @functools.partial(
        pl.kernel,
        mesh=mesh,
        out_type=jax.ShapeDtypeStruct((NC * ACC_ROWS, D), jnp.float32),
        scratch_types=[pltpu.VMEM((CHUNK,), jnp.int32) for _ in range(NBUF)]
        + [pltpu.VMEM((CHUNK,), jnp.int32) for _ in range(NBUF)]
        + [pltpu.VMEM((CHUNK, D), jnp.float32) for _ in range(NBUF)]
        + [pltpu.VMEM_SHARED((ACC_ROWS, D), jnp.float32)]
        + [pltpu.SemaphoreType.DMA for _ in range(3 * NBUF)],
    )
    def agg(x_hbm, src_hbm, dst_hbm, out_hbm, *rest):
        sidx = rest[:NBUF]
        didx = rest[NBUF:2 * NBUF]
        rows = rest[2 * NBUF:3 * NBUF]
        acc = rest[3 * NBUF]
        ssem = rest[3 * NBUF + 1:4 * NBUF + 1]
        dsem = rest[4 * NBUF + 1:5 * NBUF + 1]
        gsem = rest[5 * NBUF + 1:6 * NBUF + 1]
        cid = lax.axis_index("c")
        sid = lax.axis_index("s")
        wid = cid * NS + sid

        def zrow(r, carry):
            for c16 in range(D // 16):
                rows[0][r, pl.ds(c16 * 16, 16)] = jnp.zeros((16,), jnp.float32)
            return carry

        lax.fori_loop(0, CHUNK, zrow, 0)
        base_row = sid * TILE_STRIPE
        off = 0
        for n in STRIPE_COPIES:
            pltpu.sync_copy(rows[0].at[pl.ds(0, n)],
                            acc.at[pl.ds(base_row + off, n)])
            off += n
        plsc.subcore_barrier()

        chunk0 = wid * BASE_CHUNKS + jnp.minimum(wid, EXTRA_TILES)

        def issue_src_idx(b, c):
            eoff = pl.multiple_of((chunk0 + c) * CHUNK, 8)
            pltpu.async_copy(src_hbm.at[pl.ds(eoff, CHUNK)], sidx[b], ssem[b])

        def issue_dst_idx(b, c):
            eoff = pl.multiple_of((chunk0 + c) * CHUNK, 8)
            pltpu.async_copy(dst_hbm.at[pl.ds(eoff, CHUNK)], didx[b], dsem[b])

        def issue_idx(b, c):
            issue_src_idx(b, c)
            issue_dst_idx(b, c)

        def wait_src(b):
            pltpu.make_async_copy(src_hbm.at[pl.ds(0, CHUNK)], sidx[b], ssem[b]).wait()

        def wait_dst(b):
            pltpu.make_async_copy(dst_hbm.at[pl.ds(0, CHUNK)], didx[b], dsem[b]).wait()

        def issue_gather(b):
            pltpu.async_copy(x_hbm.at[sidx[b]], rows[b], gsem[b])

        def wait_gather(b):
            pltpu.make_async_copy(x_hbm.at[sidx[b]], rows[b], gsem[b]).wait()

        for b in range(NBUF):
            issue_idx(b, b)
        for b in range(NBUF):
            wait_src(b)
            issue_gather(b)

        def group(g, carry):
            for b in range(NBUF):
                more = g < N_GROUPS - 1
                wait_gather(b)

                @pl.when(more)
                def _():
                    issue_src_idx(b, g * NBUF + b + NBUF)

                wait_dst(b)
                pltpu.sync_copy(rows[b], acc.at[didx[b]], add=True)

                @pl.when(more)
                def _():
                    issue_dst_idx(b, g * NBUF + b + NBUF)
                    wait_src(b)
                    issue_gather(b)
            return carry

        lax.fori_loop(0, N_GROUPS, group, 0)

        @pl.when(wid < EXTRA_TILES)
        def _():
            issue_idx(0, BASE_CHUNKS)
            wait_src(0)
            issue_gather(0)
            wait_gather(0)
            wait_dst(0)
            pltpu.sync_copy(rows[0], acc.at[didx[0]], add=True)

        plsc.subcore_barrier()

        out_base = cid * ACC_ROWS + base_row
        off = 0
        for n in STRIPE_COPIES:
            pltpu.sync_copy(acc.at[pl.ds(base_row + off, n)], rows[0].at[pl.ds(0, n)])
            pltpu.sync_copy(rows[0].at[pl.ds(0, n)], out_hbm.at[pl.ds(out_base + off, n)])
            off += n

    return agg(x, src, dst)


def _tc_mlp(partials, x, epsilon, W1, b1, g1, be1, W2, b2, g2, be2):
    def body(p_ref, x_ref, eps_ref, W1_ref, b1_ref, g1_ref, be1_ref,
             W2_ref, b2_ref, g2_ref, be2_ref, out_ref):
        v = (p_ref[0:N_NODES, :] + p_ref[ACC_ROWS:ACC_ROWS + N_NODES, :]
             + eps_ref[0, 0] * x_ref[...])
        h = jnp.dot(v, W1_ref[...], preferred_element_type=jnp.float32) + b1_ref[...]
        m = jnp.mean(h, axis=0)
        var = jnp.mean((h - m) * (h - m), axis=0)
        h = jnp.maximum((h - m) * lax.rsqrt(var + 1e-5) * g1_ref[...] + be1_ref[...], 0.0)
        h = jnp.dot(h, W2_ref[...], preferred_element_type=jnp.float32) + b2_ref[...]
        m2 = jnp.mean(h, axis=0)
        var2 = jnp.mean((h - m2) * (h - m2), axis=0)
        out_ref[...] = jnp.maximum(
            (h - m2) * lax.rsqrt(var2 + 1e-5) * g2_ref[...] + be2_ref[...], 0.0)

    return pl.pallas_call(
        body,
        out_shape=jax.ShapeDtypeStruct((N_NODES, D), jnp.float32),
    )(partials, x, epsilon, W1, b1, g1, be1, W2, b2, g2, be2)


def kernel(x, edge_index, epsilon, W1, b1, g1, be1, W2, b2, g2, be2):
    src = edge_index[0]
    dst = edge_index[1]
    partials = _sc_aggregate(x, src, dst)
    return _tc_mlp(partials, x, epsilon, W1, b1, g1, be1, W2, b2, g2, be2)


# speedup vs baseline: 15.5293x; 1.1012x over previous
import functools

import jax
import jax.numpy as jnp
from jax import lax
from jax.experimental import pallas as pl
from jax.experimental.pallas import tpu as pltpu
from jax.experimental.pallas import tpu_sc as plsc

N_NODES = 10000
N_EDGES = 320000
D = 128

NC = 2
NS = 16
NW = NC * NS

CHUNK = 128
N_CHUNKS = N_EDGES // CHUNK
BASE_CHUNKS = N_CHUNKS // NW
EXTRA_TILES = N_CHUNKS - BASE_CHUNKS * NW
NBUF = 3
N_GROUPS = BASE_CHUNKS // NBUF
ACC_ROWS = 10112
TILE_STRIPE = ACC_ROWS // NS
STRIPE_COPIES = (128, 128, 128, 128, 120)


def _sc_aggregate(x, edge_index):
    mesh = plsc.VectorSubcoreMesh(core_axis_name="c", subcore_axis_name="s")

    @functools.partial(
        pl.kernel,
        mesh=mesh,
        out_type=jax.ShapeDtypeStruct((NC * ACC_ROWS, D), jnp.float32),
        scratch_types=[pltpu.VMEM((CHUNK,), jnp.int32) for _ in range(NBUF)]
        + [pltpu.VMEM((CHUNK,), jnp.int32) for _ in range(NBUF)]
        + [pltpu.VMEM((CHUNK, D), jnp.float32) for _ in range(NBUF)]
        + [pltpu.VMEM_SHARED((ACC_ROWS, D), jnp.float32)]
        + [pltpu.SemaphoreType.DMA for _ in range(3 * NBUF)],
    )
    def agg(x_hbm, ei_hbm, out_hbm, *rest):
        sidx = rest[:NBUF]
        didx = rest[NBUF:2 * NBUF]
        rows = rest[2 * NBUF:3 * NBUF]
        acc = rest[3 * NBUF]
        ssem = rest[3 * NBUF + 1:4 * NBUF + 1]
        dsem = rest[4 * NBUF + 1:5 * NBUF + 1]
        gsem = rest[5 * NBUF + 1:6 * NBUF + 1]
        cid = lax.axis_index("c")
        sid = lax.axis_index("s")
        wid = cid * NS + sid

        def zrow(r, carry):
            for c16 in range(D // 16):
                rows[0][r, pl.ds(c16 * 16, 16)] = jnp.zeros((16,), jnp.float32)
            return carry

        lax.fori_loop(0, CHUNK, zrow, 0)
        base_row = sid * TILE_STRIPE
        off = 0
        for n in STRIPE_COPIES:
            pltpu.sync_copy(rows[0].at[pl.ds(0, n)],
                            acc.at[pl.ds(base_row + off, n)])
            off += n
        plsc.subcore_barrier()

        chunk0 = wid * BASE_CHUNKS + jnp.minimum(wid, EXTRA_TILES)

        def issue_src_idx(b, c):
            eoff = pl.multiple_of((chunk0 + c) * CHUNK, 8)
            pltpu.async_copy(ei_hbm.at[0, pl.ds(eoff, CHUNK)], sidx[b], ssem[b])

        def issue_dst_idx(b, c):
            eoff = pl.multiple_of((chunk0 + c) * CHUNK, 8)
            pltpu.async_copy(ei_hbm.at[1, pl.ds(eoff, CHUNK)], didx[b], dsem[b])

        def issue_idx(b, c):
            issue_src_idx(b, c)
            issue_dst_idx(b, c)

        def wait_src(b):
            pltpu.make_async_copy(ei_hbm.at[0, pl.ds(0, CHUNK)], sidx[b], ssem[b]).wait()

        def wait_dst(b):
            pltpu.make_async_copy(ei_hbm.at[1, pl.ds(0, CHUNK)], didx[b], dsem[b]).wait()

        def issue_gather(b):
            pltpu.async_copy(x_hbm.at[sidx[b]], rows[b], gsem[b])

        def wait_gather(b):
            pltpu.make_async_copy(x_hbm.at[sidx[b]], rows[b], gsem[b]).wait()

        for b in range(NBUF):
            issue_idx(b, b)
        for b in range(NBUF):
            wait_src(b)
            issue_gather(b)

        def group(g, carry):
            for b in range(NBUF):
                more = g < N_GROUPS - 1
                wait_gather(b)

                @pl.when(more)
                def _():
                    issue_src_idx(b, g * NBUF + b + NBUF)

                wait_dst(b)
                pltpu.sync_copy(rows[b], acc.at[didx[b]], add=True)

                @pl.when(more)
                def _():
                    issue_dst_idx(b, g * NBUF + b + NBUF)
                    wait_src(b)
                    issue_gather(b)
            return carry

        lax.fori_loop(0, N_GROUPS, group, 0)

        @pl.when(wid < EXTRA_TILES)
        def _():
            issue_idx(0, BASE_CHUNKS)
            wait_src(0)
            issue_gather(0)
            wait_gather(0)
            wait_dst(0)
            pltpu.sync_copy(rows[0], acc.at[didx[0]], add=True)

        plsc.subcore_barrier()

        out_base = cid * ACC_ROWS + base_row
        off = 0
        for n in STRIPE_COPIES:
            pltpu.sync_copy(acc.at[pl.ds(base_row + off, n)], rows[0].at[pl.ds(0, n)])
            pltpu.sync_copy(rows[0].at[pl.ds(0, n)], out_hbm.at[pl.ds(out_base + off, n)])
            off += n

    return agg(x, edge_index)


def _tc_mlp(partials, x, epsilon, W1, b1, g1, be1, W2, b2, g2, be2):
    def body(p_ref, x_ref, eps_ref, W1_ref, b1_ref, g1_ref, be1_ref,
             W2_ref, b2_ref, g2_ref, be2_ref, out_ref):
        v = (p_ref[0:N_NODES, :] + p_ref[ACC_ROWS:ACC_ROWS + N_NODES, :]
             + eps_ref[0, 0] * x_ref[...])
        h = jnp.dot(v, W1_ref[...], preferred_element_type=jnp.float32) + b1_ref[...]
        m = jnp.mean(h, axis=0)
        var = jnp.mean((h - m) * (h - m), axis=0)
        h = jnp.maximum((h - m) * lax.rsqrt(var + 1e-5) * g1_ref[...] + be1_ref[...], 0.0)
        h = jnp.dot(h, W2_ref[...], preferred_element_type=jnp.float32) + b2_ref[...]
        m2 = jnp.mean(h, axis=0)
        var2 = jnp.mean((h - m2) * (h - m2), axis=0)
        out_ref[...] = jnp.maximum(
            (h - m2) * lax.rsqrt(var2 + 1e-5) * g2_ref[...] + be2_ref[...], 0.0)

    return pl.pallas_call(
        body,
        out_shape=jax.ShapeDtypeStruct((N_NODES, D), jnp.float32),
    )(partials, x, epsilon, W1, b1, g1, be1, W2, b2, g2, be2)


def kernel(x, edge_index, epsilon, W1, b1, g1, be1, W2, b2, g2, be2):
    partials = _sc_aggregate(x, edge_index)
    return _tc_mlp(partials, x, epsilon, W1, b1, g1, be1, W2, b2, g2, be2)
